# baseline (speedup 1.0000x reference)
import functools

import jax
import jax.numpy as jnp
from jax.experimental import pallas as pl
from jax.experimental.pallas import tpu as pltpu

_N = 10000
_NE = 320000
_D = 128
_ROWS = 1000
_GRID = _N // _ROWS


def _finish_body(s_ref, w_ref, cnt_ref, h_ref, bf_ref, bb_ref, g_ref, be_ref,
                 out_ref):
    acc = jnp.zeros((_ROWS, _D), dtype=jnp.float32)
    for j in range(12):
        acc += jnp.dot(s_ref[j], w_ref[j], preferred_element_type=jnp.float32)
    cf = jnp.sum(cnt_ref[:, :32], axis=1)
    cb = jnp.sum(cnt_ref[:, 32:], axis=1)
    acc = acc + cf[:, None] * bf_ref[0][None, :] + cb[:, None] * bb_ref[0][None, :]
    cnt = cf + cb
    agg = acc / jnp.maximum(cnt, 1.0)[:, None]
    act = jnp.where(agg >= 0, agg, 0.01 * agg)
    x = act + h_ref[...]
    mean = jnp.mean(x, axis=1, keepdims=True)
    var = jnp.mean((x - mean) ** 2, axis=1, keepdims=True)
    out_ref[...] = (x - mean) * jax.lax.rsqrt(var + 1e-5) * g_ref[0][None, :] \
        + be_ref[0][None, :]


def _finish(S_all, W_all, CNT, H, b_fwd, b_back, ln_gamma, ln_beta):
    return pl.pallas_call(
        _finish_body,
        grid=(_GRID,),
        in_specs=[
            pl.BlockSpec((12, _ROWS, 64), lambda i: (0, i, 0)),
            pl.BlockSpec((12, 64, _D), lambda i: (0, 0, 0)),
            pl.BlockSpec((_ROWS, 64), lambda i: (i, 0)),
            pl.BlockSpec((_ROWS, _D), lambda i: (i, 0)),
            pl.BlockSpec((1, _D), lambda i: (0, 0)),
            pl.BlockSpec((1, _D), lambda i: (0, 0)),
            pl.BlockSpec((1, _D), lambda i: (0, 0)),
            pl.BlockSpec((1, _D), lambda i: (0, 0)),
        ],
        out_specs=pl.BlockSpec((_ROWS, _D), lambda i: (i, 0)),
        out_shape=jax.ShapeDtypeStruct((_N, _D), jnp.float32),
    )(S_all, W_all, CNT, H, b_fwd.reshape(1, _D), b_back.reshape(1, _D),
      ln_gamma.reshape(1, _D), ln_beta.reshape(1, _D))


def _edge_pass_jnp(H, E, h0, h1):
    S = []
    CNT = []
    ones = jnp.ones((_NE,), dtype=jnp.float32)
    for g_idx, s_idx in ((h0, h1), (h1, h0)):
        G = jnp.take(H, g_idx, axis=0)
        S.append(jax.ops.segment_sum(G, s_idx, num_segments=_N))
        S.append(jax.ops.segment_sum(E, s_idx, num_segments=_N))
        S.append(jax.ops.segment_sum(G * E, s_idx, num_segments=_N))
        CNT.append(jax.ops.segment_sum(ones, s_idx, num_segments=_N))
    S = jnp.stack(S)
    S_all = jnp.concatenate([S[:, :, :64], S[:, :, 64:]], axis=0)
    CNT = jnp.stack(CNT).reshape(2, 1, _N)
    CNT = jnp.broadcast_to(CNT / 32.0, (2, 32, _N))
    return S_all, CNT.reshape(64, _N).T


def kernel(H, E, ht, W_fwd, b_fwd, W_back, b_back, ln_gamma, ln_beta):
    h0 = ht[:, 0].astype(jnp.int32)
    h1 = ht[:, 1].astype(jnp.int32)

    def wsplit(W):
        WT = W.T
        return WT[0:128] + WT[256:384], WT[128:256] + WT[256:384], WT[384:512]

    Whf, Wef, W4f = wsplit(W_fwd)
    Whb, Web, W4b = wsplit(W_back)
    Wmats = [Whf, Wef, W4f, Whb, Web, W4b]
    W_all = jnp.stack([m[:64] for m in Wmats] + [m[64:] for m in Wmats])

    S_all, CNT = _edge_pass_jnp(H, E, h0, h1)
    return _finish(S_all, W_all, CNT, H, b_fwd, b_back, ln_gamma, ln_beta)


# speedup vs baseline: 1.7717x; 1.7717x over previous
import functools

import jax
import jax.numpy as jnp
from jax import lax
from jax.experimental import pallas as pl
from jax.experimental.pallas import tpu as pltpu
from jax.experimental.pallas import tpu_sc as plsc

_N = 10000
_NE = 320000
_D = 128
_WIDTHS = (32, 32, 32, 32)
_ROWS = 1000
_GRID = _N // _ROWS

_NP = 10240
_NT = 16
_EPT = _NE // _NT
_C = 80
_NCHUNK = _EPT // _C
_RPT = _NP // _NT


def _edge_body(w, idx_hbm, h_hbm, e_hbm, s_out,
               gidx_v, sidx_v, e_v, g_v, p_v, z_v, s1_sh, s2_sh, s3_sh,
               sem):
    cid = lax.axis_index("c")
    tid = lax.axis_index("s")
    zero16 = jnp.zeros((16,), jnp.float32)

    def zrow(r, carry):
        for f in range(w // 16):
            z_v[r, pl.ds(16 * f, 16)] = zero16
        return carry
    lax.fori_loop(0, _RPT, zrow, 0)
    nbase = tid * _RPT
    pltpu.sync_copy(z_v, s1_sh.at[pl.ds(nbase, _RPT)])
    pltpu.sync_copy(z_v, s2_sh.at[pl.ds(nbase, _RPT)])
    pltpu.sync_copy(z_v, s3_sh.at[pl.ds(nbase, _RPT)])
    plsc.subcore_barrier()

    ebase0 = tid * _EPT

    def chunk(c, carry):
        eb = ebase0 + c * _C
        pltpu.sync_copy(idx_hbm.at[pl.ds(2 * cid * _NE + eb, _C)], gidx_v)
        pltpu.sync_copy(idx_hbm.at[pl.ds((2 * cid + 1) * _NE + eb, _C)],
                        sidx_v)
        pltpu.sync_copy(e_hbm.at[pl.ds(eb, _C)], e_v)
        pltpu.async_copy(h_hbm.at[gidx_v], g_v, sem).wait()

        def prow(r, c2):
            for f in range(w // 16):
                sl = pl.ds(16 * f, 16)
                p_v[r, sl] = g_v[r, sl] * e_v[r, sl]
            return c2
        lax.fori_loop(0, _C, prow, 0)
        pltpu.sync_copy(g_v, s1_sh.at[sidx_v], add=True)
        pltpu.sync_copy(e_v, s2_sh.at[sidx_v], add=True)
        pltpu.sync_copy(p_v, s3_sh.at[sidx_v], add=True)
        return carry
    lax.fori_loop(0, _NCHUNK, chunk, 0)
    plsc.subcore_barrier()

    pltpu.sync_copy(s1_sh.at[pl.ds(nbase, _RPT)],
                    s_out.at[cid, pl.ds(nbase, _RPT)])
    pltpu.sync_copy(s2_sh.at[pl.ds(nbase, _RPT)],
                    s_out.at[2 + cid, pl.ds(nbase, _RPT)])
    pltpu.sync_copy(s3_sh.at[pl.ds(nbase, _RPT)],
                    s_out.at[4 + cid, pl.ds(nbase, _RPT)])


def _edge_pass_sc(IDX, Hcols, Ecols, w):
    scratch = [
        pltpu.VMEM((_C,), jnp.int32),
        pltpu.VMEM((_C,), jnp.int32),
        pltpu.VMEM((_C, w), jnp.float32),
        pltpu.VMEM((_C, w), jnp.float32),
        pltpu.VMEM((_C, w), jnp.float32),
        pltpu.VMEM((_RPT, w), jnp.float32),
        pltpu.VMEM_SHARED((_NP, w), jnp.float32),
        pltpu.VMEM_SHARED((_NP, w), jnp.float32),
        pltpu.VMEM_SHARED((_NP, w), jnp.float32),
        pltpu.SemaphoreType.DMA,
    ]
    mesh = plsc.VectorSubcoreMesh(core_axis_name="c", subcore_axis_name="s")
    return pl.kernel(
        functools.partial(_edge_body, w),
        out_type=jax.ShapeDtypeStruct((6, _NP, w), jnp.float32),
        mesh=mesh,
        scratch_types=scratch,
        compiler_params=pltpu.CompilerParams(use_tc_tiling_on_sc=False),
    )(IDX, Hcols, Ecols)


def _cnt_body(idx_hbm, cnt_out, sidx_v, ones_v, z_v, cnt_sh, sem):
    cid = lax.axis_index("c")
    tid = lax.axis_index("s")
    zero16 = jnp.zeros((16,), jnp.float32)
    ones16 = jnp.ones((16,), jnp.float32)

    def zrow(r, carry):
        z_v[r, pl.ds(0, 16)] = zero16
        ones_v[r % _C, pl.ds(0, 16)] = ones16
        return carry
    lax.fori_loop(0, _RPT, zrow, 0)
    nbase = tid * _RPT
    pltpu.sync_copy(z_v, cnt_sh.at[pl.ds(nbase, _RPT)])
    plsc.subcore_barrier()

    ebase0 = tid * _EPT

    def chunk(c, carry):
        eb = ebase0 + c * _C
        pltpu.sync_copy(idx_hbm.at[pl.ds((2 * cid + 1) * _NE + eb, _C)],
                        sidx_v)
        pltpu.sync_copy(ones_v, cnt_sh.at[sidx_v], add=True)
        return carry
    lax.fori_loop(0, _NCHUNK, chunk, 0)
    plsc.subcore_barrier()
    pltpu.sync_copy(cnt_sh.at[pl.ds(nbase, _RPT)],
                    cnt_out.at[cid, pl.ds(nbase, _RPT)])


def _cnt_pass_sc(IDX):
    scratch = [
        pltpu.VMEM((_C,), jnp.int32),
        pltpu.VMEM((_C, 16), jnp.float32),
        pltpu.VMEM((_RPT, 16), jnp.float32),
        pltpu.VMEM_SHARED((_NP, 16), jnp.float32),
        pltpu.SemaphoreType.DMA,
    ]
    mesh = plsc.VectorSubcoreMesh(core_axis_name="c", subcore_axis_name="s")
    return pl.kernel(
        _cnt_body,
        out_type=jax.ShapeDtypeStruct((2, _NP, 16), jnp.float32),
        mesh=mesh,
        scratch_types=scratch,
        compiler_params=pltpu.CompilerParams(use_tc_tiling_on_sc=False),
    )(IDX)


def _finish_body(*refs):
    ng = len(_WIDTHS)
    s_refs = refs[:ng]
    w_refs = refs[ng:2 * ng]
    (cf_ref, cb_ref, h_ref, bf_ref, bb_ref, g_ref, be_ref, out_ref) =         refs[2 * ng:]
    acc = jnp.zeros((_ROWS, _D), dtype=jnp.float32)
    for j in range(6):
        for sr, wr in zip(s_refs, w_refs):
            acc += jnp.dot(sr[j], wr[j], preferred_element_type=jnp.float32)
    cf = cf_ref[:, 0]
    cb = cb_ref[:, 0]
    acc = acc + cf[:, None] * bf_ref[0][None, :] + cb[:, None] * bb_ref[0][None, :]
    cnt = cf + cb
    agg = acc / jnp.maximum(cnt, 1.0)[:, None]
    act = jnp.where(agg >= 0, agg, 0.01 * agg)
    x = act + h_ref[...]
    mean = jnp.mean(x, axis=1, keepdims=True)
    var = jnp.mean((x - mean) ** 2, axis=1, keepdims=True)
    out_ref[...] = (x - mean) * jax.lax.rsqrt(var + 1e-5) * g_ref[0][None, :] \
        + be_ref[0][None, :]


def _finish(S3, W3, CNTF, CNTB, H, b_fwd, b_back, ln_gamma, ln_beta):
    s_specs = [pl.BlockSpec((6, _ROWS, w), lambda i: (0, i, 0))
               for w in _WIDTHS]
    w_specs = [pl.BlockSpec((6, w, _D), lambda i: (0, 0, 0))
               for w in _WIDTHS]
    return pl.pallas_call(
        _finish_body,
        grid=(_GRID,),
        in_specs=s_specs + w_specs + [
            pl.BlockSpec((_ROWS, 16), lambda i: (i, 0)),
            pl.BlockSpec((_ROWS, 16), lambda i: (i, 0)),
            pl.BlockSpec((_ROWS, _D), lambda i: (i, 0)),
            pl.BlockSpec((1, _D), lambda i: (0, 0)),
            pl.BlockSpec((1, _D), lambda i: (0, 0)),
            pl.BlockSpec((1, _D), lambda i: (0, 0)),
            pl.BlockSpec((1, _D), lambda i: (0, 0)),
        ],
        out_specs=pl.BlockSpec((_ROWS, _D), lambda i: (i, 0)),
        out_shape=jax.ShapeDtypeStruct((_N, _D), jnp.float32),
    )(*S3, *W3, CNTF, CNTB, H,
      b_fwd.reshape(1, _D), b_back.reshape(1, _D), ln_gamma.reshape(1, _D),
      ln_beta.reshape(1, _D))


def kernel(H, E, ht, W_fwd, b_fwd, W_back, b_back, ln_gamma, ln_beta):
    h0 = ht[:, 0].astype(jnp.int32)
    h1 = ht[:, 1].astype(jnp.int32)
    IDX = jnp.concatenate([h0, h1, h1, h0])

    def wsplit(W):
        WT = W.T
        return WT[0:128] + WT[256:384], WT[128:256] + WT[256:384], WT[384:512]

    Whf, Wef, W4f = wsplit(W_fwd)
    Whb, Web, W4b = wsplit(W_back)
    mats = [Whf, Whb, Wef, Web, W4f, W4b]

    S3, W3 = [], []
    off = 0
    for w in _WIDTHS:
        W3.append(jnp.stack([m[off:off + w] for m in mats]))
        S3.append(_edge_pass_sc(IDX, H[:, off:off + w], E[:, off:off + w], w))
        off += w
    CNT = _cnt_pass_sc(IDX)

    return _finish(S3, W3, CNT[0, :_N], CNT[1, :_N], H,
                   b_fwd, b_back, ln_gamma, ln_beta)


# speedup vs baseline: 4.7212x; 2.6648x over previous
import functools

import jax
import jax.numpy as jnp
from jax import lax
from jax.experimental import pallas as pl
from jax.experimental.pallas import tpu as pltpu
from jax.experimental.pallas import tpu_sc as plsc

_N = 10000
_NE = 320000
_D = 128
_WIDTHS = (32, 32, 32, 32)
_ROWS = 1000
_GRID = _N // _ROWS

_NP = 10240
_NT = 16
_EPT = _NE // _NT
_C = 80
_NCHUNK = _EPT // _C
_RPT = _NP // _NT


def _edge_body(w, idx_hbm, h_hbm, e_hbm, s_out,
               gidx_v, sidx_v, e_v, g_v, p_v, z_v, s1_sh, s2_sh, s3_sh,
               *sems):
    lsems, gsems, ssems = sems[0:5], sems[5:10], sems[10:15]
    cid = lax.axis_index("c")
    tid = lax.axis_index("s")
    zero16 = jnp.zeros((16,), jnp.float32)

    def zrow(r, carry):
        for f in range(w // 16):
            z_v[r, pl.ds(16 * f, 16)] = zero16
        return carry
    lax.fori_loop(0, _RPT, zrow, 0)
    nbase = tid * _RPT
    pltpu.sync_copy(z_v, s1_sh.at[pl.ds(nbase, _RPT)])
    pltpu.sync_copy(z_v, s2_sh.at[pl.ds(nbase, _RPT)])
    pltpu.sync_copy(z_v, s3_sh.at[pl.ds(nbase, _RPT)])
    plsc.subcore_barrier()

    ebase0 = tid * _EPT
    gb = 2 * cid * _NE
    sb = (2 * cid + 1) * _NE

    def load_descs(c, sl):
        eb = ebase0 + c * _C
        return (
            pltpu.make_async_copy(idx_hbm.at[pl.ds(gb + eb, _C)],
                                  gidx_v.at[sl], lsems[sl]),
            pltpu.make_async_copy(idx_hbm.at[pl.ds(sb + eb, _C)],
                                  sidx_v.at[sl], lsems[sl]),
            pltpu.make_async_copy(e_hbm.at[pl.ds(eb, _C)], e_v.at[sl],
                                  lsems[sl]),
        )

    def gather_desc(sl):
        return pltpu.make_async_copy(h_hbm.at[gidx_v.at[sl]], g_v.at[sl],
                                     gsems[sl])

    def scat_descs(sl):
        return (
            pltpu.make_async_copy(g_v.at[sl], s1_sh.at[sidx_v.at[sl]],
                                  ssems[sl]),
            pltpu.make_async_copy(e_v.at[sl], s2_sh.at[sidx_v.at[sl]],
                                  ssems[sl]),
            pltpu.make_async_copy(p_v.at[sl], s3_sh.at[sidx_v.at[sl]],
                                  ssems[sl]),
        )

    def issue_loads(c, sl):
        for d in load_descs(c, sl):
            d.start()

    def wait_loads(c, sl):
        for d in load_descs(c, sl):
            d.wait()

    def issue_scats(sl):
        for d in scat_descs(sl):
            d.start(add=True)

    def wait_scats(sl):
        for d in scat_descs(sl):
            d.wait()

    issue_loads(0, 0)
    issue_loads(1, 1)
    issue_loads(2, 2)
    wait_loads(0, 0)
    gather_desc(0).start()
    wait_loads(1, 1)
    gather_desc(1).start()

    def macro(m, carry):
        for s_pos in range(5):
            c = m * 5 + s_pos
            sw = (s_pos + 3) % 5
            sg = (s_pos + 2) % 5

            @pl.when(c >= 2)
            def _():
                wait_scats(sw)

            @pl.when(c + 3 < _NCHUNK)
            def _():
                issue_loads(c + 3, sw)

            @pl.when(c + 2 < _NCHUNK)
            def _():
                wait_loads(c + 2, sg)
                gather_desc(sg).start()

            gather_desc(s_pos).wait()

            def prow(r, c2):
                for f in range(w // 16):
                    sl_ = pl.ds(16 * f, 16)
                    p_v[s_pos, r, sl_] = g_v[s_pos, r, sl_] * e_v[s_pos, r, sl_]
                return c2
            lax.fori_loop(0, _C, prow, 0)
            issue_scats(s_pos)
        return carry
    lax.fori_loop(0, _NCHUNK // 5, macro, 0)
    wait_scats((_NCHUNK - 2) % 5)
    wait_scats((_NCHUNK - 1) % 5)
    plsc.subcore_barrier()

    pltpu.sync_copy(s1_sh.at[pl.ds(nbase, _RPT)],
                    s_out.at[cid, pl.ds(nbase, _RPT)])
    pltpu.sync_copy(s2_sh.at[pl.ds(nbase, _RPT)],
                    s_out.at[2 + cid, pl.ds(nbase, _RPT)])
    pltpu.sync_copy(s3_sh.at[pl.ds(nbase, _RPT)],
                    s_out.at[4 + cid, pl.ds(nbase, _RPT)])


def _edge_pass_sc(IDX, Hcols, Ecols, w):
    scratch = [
        pltpu.VMEM((5, _C), jnp.int32),
        pltpu.VMEM((5, _C), jnp.int32),
        pltpu.VMEM((5, _C, w), jnp.float32),
        pltpu.VMEM((5, _C, w), jnp.float32),
        pltpu.VMEM((5, _C, w), jnp.float32),
        pltpu.VMEM((_RPT, w), jnp.float32),
        pltpu.VMEM_SHARED((_NP, w), jnp.float32),
        pltpu.VMEM_SHARED((_NP, w), jnp.float32),
        pltpu.VMEM_SHARED((_NP, w), jnp.float32),
    ] + [pltpu.SemaphoreType.DMA] * 15
    mesh = plsc.VectorSubcoreMesh(core_axis_name="c", subcore_axis_name="s")
    return pl.kernel(
        functools.partial(_edge_body, w),
        out_type=jax.ShapeDtypeStruct((6, _NP, w), jnp.float32),
        mesh=mesh,
        scratch_types=scratch,
        compiler_params=pltpu.CompilerParams(use_tc_tiling_on_sc=False),
    )(IDX, Hcols, Ecols)


def _cnt_body(idx_hbm, cnt_out, sidx_v, ones_v, z_v, cnt_sh, sem):
    cid = lax.axis_index("c")
    tid = lax.axis_index("s")
    zero16 = jnp.zeros((16,), jnp.float32)
    ones16 = jnp.ones((16,), jnp.float32)

    def zrow(r, carry):
        z_v[r, pl.ds(0, 16)] = zero16
        ones_v[r % _C, pl.ds(0, 16)] = ones16
        return carry
    lax.fori_loop(0, _RPT, zrow, 0)
    nbase = tid * _RPT
    pltpu.sync_copy(z_v, cnt_sh.at[pl.ds(nbase, _RPT)])
    plsc.subcore_barrier()

    ebase0 = tid * _EPT

    def chunk(c, carry):
        eb = ebase0 + c * _C
        pltpu.sync_copy(idx_hbm.at[pl.ds((2 * cid + 1) * _NE + eb, _C)],
                        sidx_v)
        pltpu.sync_copy(ones_v, cnt_sh.at[sidx_v], add=True)
        return carry
    lax.fori_loop(0, _NCHUNK, chunk, 0)
    plsc.subcore_barrier()
    pltpu.sync_copy(cnt_sh.at[pl.ds(nbase, _RPT)],
                    cnt_out.at[cid, pl.ds(nbase, _RPT)])


def _cnt_pass_sc(IDX):
    scratch = [
        pltpu.VMEM((_C,), jnp.int32),
        pltpu.VMEM((_C, 16), jnp.float32),
        pltpu.VMEM((_RPT, 16), jnp.float32),
        pltpu.VMEM_SHARED((_NP, 16), jnp.float32),
        pltpu.SemaphoreType.DMA,
    ]
    mesh = plsc.VectorSubcoreMesh(core_axis_name="c", subcore_axis_name="s")
    return pl.kernel(
        _cnt_body,
        out_type=jax.ShapeDtypeStruct((2, _NP, 16), jnp.float32),
        mesh=mesh,
        scratch_types=scratch,
        compiler_params=pltpu.CompilerParams(use_tc_tiling_on_sc=False),
    )(IDX)


def _finish_body(*refs):
    ng = len(_WIDTHS)
    s_refs = refs[:ng]
    w_refs = refs[ng:2 * ng]
    (cf_ref, cb_ref, h_ref, bf_ref, bb_ref, g_ref, be_ref, out_ref) =         refs[2 * ng:]
    acc = jnp.zeros((_ROWS, _D), dtype=jnp.float32)
    for j in range(6):
        for sr, wr in zip(s_refs, w_refs):
            acc += jnp.dot(sr[j], wr[j], preferred_element_type=jnp.float32)
    cf = cf_ref[:, 0]
    cb = cb_ref[:, 0]
    acc = acc + cf[:, None] * bf_ref[0][None, :] + cb[:, None] * bb_ref[0][None, :]
    cnt = cf + cb
    agg = acc / jnp.maximum(cnt, 1.0)[:, None]
    act = jnp.where(agg >= 0, agg, 0.01 * agg)
    x = act + h_ref[...]
    mean = jnp.mean(x, axis=1, keepdims=True)
    var = jnp.mean((x - mean) ** 2, axis=1, keepdims=True)
    out_ref[...] = (x - mean) * jax.lax.rsqrt(var + 1e-5) * g_ref[0][None, :] \
        + be_ref[0][None, :]


def _finish(S3, W3, CNTF, CNTB, H, b_fwd, b_back, ln_gamma, ln_beta):
    s_specs = [pl.BlockSpec((6, _ROWS, w), lambda i: (0, i, 0))
               for w in _WIDTHS]
    w_specs = [pl.BlockSpec((6, w, _D), lambda i: (0, 0, 0))
               for w in _WIDTHS]
    return pl.pallas_call(
        _finish_body,
        grid=(_GRID,),
        in_specs=s_specs + w_specs + [
            pl.BlockSpec((_ROWS, 16), lambda i: (i, 0)),
            pl.BlockSpec((_ROWS, 16), lambda i: (i, 0)),
            pl.BlockSpec((_ROWS, _D), lambda i: (i, 0)),
            pl.BlockSpec((1, _D), lambda i: (0, 0)),
            pl.BlockSpec((1, _D), lambda i: (0, 0)),
            pl.BlockSpec((1, _D), lambda i: (0, 0)),
            pl.BlockSpec((1, _D), lambda i: (0, 0)),
        ],
        out_specs=pl.BlockSpec((_ROWS, _D), lambda i: (i, 0)),
        out_shape=jax.ShapeDtypeStruct((_N, _D), jnp.float32),
    )(*S3, *W3, CNTF, CNTB, H,
      b_fwd.reshape(1, _D), b_back.reshape(1, _D), ln_gamma.reshape(1, _D),
      ln_beta.reshape(1, _D))


def kernel(H, E, ht, W_fwd, b_fwd, W_back, b_back, ln_gamma, ln_beta):
    h0 = ht[:, 0].astype(jnp.int32)
    h1 = ht[:, 1].astype(jnp.int32)
    IDX = jnp.concatenate([h0, h1, h1, h0])

    def wsplit(W):
        WT = W.T
        return WT[0:128] + WT[256:384], WT[128:256] + WT[256:384], WT[384:512]

    Whf, Wef, W4f = wsplit(W_fwd)
    Whb, Web, W4b = wsplit(W_back)
    mats = [Whf, Whb, Wef, Web, W4f, W4b]

    S3, W3 = [], []
    off = 0
    for w in _WIDTHS:
        W3.append(jnp.stack([m[off:off + w] for m in mats]))
        S3.append(_edge_pass_sc(IDX, H[:, off:off + w], E[:, off:off + w], w))
        off += w
    CNT = _cnt_pass_sc(IDX)

    return _finish(S3, W3, CNT[0, :_N], CNT[1, :_N], H,
                   b_fwd, b_back, ln_gamma, ln_beta)


# speedup vs baseline: 7.0960x; 1.5030x over previous
import functools

import jax
import jax.numpy as jnp
from jax import lax
from jax.experimental import pallas as pl
from jax.experimental.pallas import tpu as pltpu
from jax.experimental.pallas import tpu_sc as plsc

_N = 10000
_NE = 320000
_D = 128
_W = 32
_NG = 4
_ROWS = 1000
_GRID = _N // _ROWS

_NP = 10240
_NT = 16
_EPT = _NE // _NT
_C = 80
_NCHUNK = _EPT // _C
_RPT = _NP // _NT


def _edge_body(idx_hbm, hs_hbm, e_hbm, s_out,
               gidx_v, sidx_v, e_v, g_v, p_v, z_v, ones_v,
               s1_sh, s2_sh, s3_sh, *sems):
    w = _W
    lsems, gsems, ssems = sems[0:5], sems[5:10], sems[10:15]
    cid = lax.axis_index("c")
    tid = lax.axis_index("s")
    zero16 = jnp.zeros((16,), jnp.float32)
    ones16 = jnp.ones((16,), jnp.float32)

    def zrow(r, carry):
        for f in range(w // 16):
            z_v[r, pl.ds(16 * f, 16)] = zero16
        return carry
    lax.fori_loop(0, _RPT, zrow, 0)

    def orow(r, carry):
        for f in range(w // 16):
            ones_v[r, pl.ds(16 * f, 16)] = ones16
        return carry
    lax.fori_loop(0, _C, orow, 0)

    nbase = tid * _RPT
    ebase0 = tid * _EPT
    gb = 2 * cid * _NE
    sb = (2 * cid + 1) * _NE

    for gi in range(_NG + 1):
        counts_only = gi == _NG
        if counts_only:
            pltpu.sync_copy(z_v, s2_sh.at[pl.ds(nbase, _RPT)])
        else:
            pltpu.sync_copy(z_v, s1_sh.at[pl.ds(nbase, _RPT)])
            pltpu.sync_copy(z_v, s2_sh.at[pl.ds(nbase, _RPT)])
            pltpu.sync_copy(z_v, s3_sh.at[pl.ds(nbase, _RPT)])
        plsc.subcore_barrier()

        def load_descs(c, sl):
            eb = ebase0 + c * _C
            descs = [
                pltpu.make_async_copy(idx_hbm.at[pl.ds(sb + eb, _C)],
                                      sidx_v.at[sl], lsems[sl]),
            ]
            if not counts_only:
                descs.append(
                    pltpu.make_async_copy(idx_hbm.at[pl.ds(gb + eb, _C)],
                                          gidx_v.at[sl], lsems[sl]))
                descs.append(
                    pltpu.make_async_copy(
                        e_hbm.at[pl.ds(eb, _C), pl.ds(gi * _W, _W)],
                        e_v.at[sl], lsems[sl]))
            return descs

        def gather_desc(sl):
            return pltpu.make_async_copy(hs_hbm.at[gidx_v.at[sl]],
                                         g_v.at[sl], gsems[sl])

        def shift_gidx(sl):
            if gi > 0:
                for f in range(_C // 16):
                    sl_ = pl.ds(16 * f, 16)
                    gidx_v[sl, sl_] = gidx_v[sl, sl_] + (gi * _N)

        def scat_descs(sl):
            if counts_only:
                return [
                    pltpu.make_async_copy(ones_v, s2_sh.at[sidx_v.at[sl]],
                                          ssems[sl]),
                ]
            return [
                pltpu.make_async_copy(g_v.at[sl], s1_sh.at[sidx_v.at[sl]],
                                      ssems[sl]),
                pltpu.make_async_copy(e_v.at[sl], s2_sh.at[sidx_v.at[sl]],
                                      ssems[sl]),
                pltpu.make_async_copy(p_v.at[sl], s3_sh.at[sidx_v.at[sl]],
                                      ssems[sl]),
            ]

        def issue_loads(c, sl):
            for d in load_descs(c, sl):
                d.start()

        def wait_loads(c, sl):
            for d in load_descs(c, sl):
                d.wait()

        def issue_scats(sl):
            for d in scat_descs(sl):
                d.start(add=True)

        def wait_scats(sl):
            for d in scat_descs(sl):
                d.wait()

        issue_loads(0, 0)
        issue_loads(1, 1)
        issue_loads(2, 2)
        wait_loads(0, 0)
        wait_loads(1, 1)
        if not counts_only:
            shift_gidx(0)
            gather_desc(0).start()
            shift_gidx(1)
            gather_desc(1).start()

        def macro(m, carry):
            for s_pos in range(5):
                c = m * 5 + s_pos
                sw = (s_pos + 3) % 5
                sg = (s_pos + 2) % 5

                @pl.when(c >= 2)
                def _():
                    wait_scats(sw)

                @pl.when(c + 3 < _NCHUNK)
                def _():
                    issue_loads(c + 3, sw)

                if counts_only:
                    @pl.when(c + 2 < _NCHUNK)
                    def _():
                        wait_loads(c + 2, sg)
                else:
                    @pl.when(c + 2 < _NCHUNK)
                    def _():
                        wait_loads(c + 2, sg)
                        shift_gidx(sg)
                        gather_desc(sg).start()

                    gather_desc(s_pos).wait()

                    def prow(r, c2):
                        for f in range(w // 16):
                            sl_ = pl.ds(16 * f, 16)
                            p_v[s_pos, r, sl_] = \
                                g_v[s_pos, r, sl_] * e_v[s_pos, r, sl_]
                        return c2
                    lax.fori_loop(0, _C, prow, 0)
                issue_scats(s_pos)
            return carry
        lax.fori_loop(0, _NCHUNK // 5, macro, 0)
        wait_scats((_NCHUNK - 2) % 5)
        wait_scats((_NCHUNK - 1) % 5)
        plsc.subcore_barrier()

        if counts_only:
            pltpu.sync_copy(s2_sh.at[pl.ds(nbase, _RPT)],
                            s_out.at[6 * _NG + cid, pl.ds(nbase, _RPT)])
        else:
            pltpu.sync_copy(s1_sh.at[pl.ds(nbase, _RPT)],
                            s_out.at[6 * gi + cid, pl.ds(nbase, _RPT)])
            pltpu.sync_copy(s2_sh.at[pl.ds(nbase, _RPT)],
                            s_out.at[6 * gi + 2 + cid, pl.ds(nbase, _RPT)])
            pltpu.sync_copy(s3_sh.at[pl.ds(nbase, _RPT)],
                            s_out.at[6 * gi + 4 + cid, pl.ds(nbase, _RPT)])


def _edge_pass_sc(IDX, Hstack, E):
    w = _W
    out_type = jax.ShapeDtypeStruct((6 * _NG + 2, _NP, w), jnp.float32)
    scratch = [
        pltpu.VMEM((5, _C), jnp.int32),
        pltpu.VMEM((5, _C), jnp.int32),
        pltpu.VMEM((5, _C, w), jnp.float32),
        pltpu.VMEM((5, _C, w), jnp.float32),
        pltpu.VMEM((5, _C, w), jnp.float32),
        pltpu.VMEM((_RPT, w), jnp.float32),
        pltpu.VMEM((_C, w), jnp.float32),
        pltpu.VMEM_SHARED((_NP, w), jnp.float32),
        pltpu.VMEM_SHARED((_NP, w), jnp.float32),
        pltpu.VMEM_SHARED((_NP, w), jnp.float32),
    ] + [pltpu.SemaphoreType.DMA] * 15
    mesh = plsc.VectorSubcoreMesh(core_axis_name="c", subcore_axis_name="s")
    return pl.kernel(
        _edge_body,
        out_type=out_type,
        mesh=mesh,
        scratch_types=scratch,
        compiler_params=pltpu.CompilerParams(use_tc_tiling_on_sc=False),
    )(IDX, Hstack, E)


def _finish_body(*refs):
    ng = _NG
    s_refs = refs[:ng]
    w_refs = refs[ng:2 * ng]
    (cf_ref, cb_ref, h_ref, bf_ref, bb_ref, g_ref, be_ref, out_ref) =         refs[2 * ng:]
    acc = jnp.zeros((_ROWS, _D), dtype=jnp.float32)
    for j in range(6):
        for sr, wr in zip(s_refs, w_refs):
            acc += jnp.dot(sr[j], wr[j], preferred_element_type=jnp.float32)
    cf = cf_ref[:, 0]
    cb = cb_ref[:, 0]
    acc = acc + cf[:, None] * bf_ref[0][None, :] + cb[:, None] * bb_ref[0][None, :]
    cnt = cf + cb
    agg = acc / jnp.maximum(cnt, 1.0)[:, None]
    act = jnp.where(agg >= 0, agg, 0.01 * agg)
    x = act + h_ref[...]
    mean = jnp.mean(x, axis=1, keepdims=True)
    var = jnp.mean((x - mean) ** 2, axis=1, keepdims=True)
    out_ref[...] = (x - mean) * jax.lax.rsqrt(var + 1e-5) * g_ref[0][None, :] \
        + be_ref[0][None, :]


def _finish(S3, W3, CNTF, CNTB, H, b_fwd, b_back, ln_gamma, ln_beta):
    s_specs = [pl.BlockSpec((6, _ROWS, _W), lambda i: (0, i, 0))
               for _ in range(_NG)]
    w_specs = [pl.BlockSpec((6, _W, _D), lambda i: (0, 0, 0))
               for _ in range(_NG)]
    return pl.pallas_call(
        _finish_body,
        grid=(_GRID,),
        in_specs=s_specs + w_specs + [
            pl.BlockSpec((_ROWS, _W), lambda i: (i, 0)),
            pl.BlockSpec((_ROWS, _W), lambda i: (i, 0)),
            pl.BlockSpec((_ROWS, _D), lambda i: (i, 0)),
            pl.BlockSpec((1, _D), lambda i: (0, 0)),
            pl.BlockSpec((1, _D), lambda i: (0, 0)),
            pl.BlockSpec((1, _D), lambda i: (0, 0)),
            pl.BlockSpec((1, _D), lambda i: (0, 0)),
        ],
        out_specs=pl.BlockSpec((_ROWS, _D), lambda i: (i, 0)),
        out_shape=jax.ShapeDtypeStruct((_N, _D), jnp.float32),
    )(*S3, *W3, CNTF, CNTB, H,
      b_fwd.reshape(1, _D), b_back.reshape(1, _D), ln_gamma.reshape(1, _D),
      ln_beta.reshape(1, _D))


def kernel(H, E, ht, W_fwd, b_fwd, W_back, b_back, ln_gamma, ln_beta):
    h0 = ht[:, 0].astype(jnp.int32)
    h1 = ht[:, 1].astype(jnp.int32)
    IDX = jnp.concatenate([h0, h1, h1, h0])

    def wsplit(W):
        WT = W.T
        return WT[0:128] + WT[256:384], WT[128:256] + WT[256:384], WT[384:512]

    Whf, Wef, W4f = wsplit(W_fwd)
    Whb, Web, W4b = wsplit(W_back)
    mats = [Whf, Whb, Wef, Web, W4f, W4b]

    W3 = [jnp.stack([m[gi * _W:(gi + 1) * _W] for m in mats])
          for gi in range(_NG)]
    Hstack = jnp.concatenate([H[:, gi * _W:(gi + 1) * _W]
                              for gi in range(_NG)])
    Sstack = _edge_pass_sc(IDX, Hstack, E)
    S3 = [Sstack[6 * gi:6 * (gi + 1)] for gi in range(_NG)]

    return _finish(S3, W3, Sstack[6 * _NG, :_N], Sstack[6 * _NG + 1, :_N],
                   H, b_fwd, b_back, ln_gamma, ln_beta)
